# baseline (speedup 1.0000x reference)
import functools

import jax
import jax.numpy as jnp
from jax import lax
from jax.experimental import pallas as pl
from jax.experimental.pallas import tpu as pltpu
from jax.experimental.pallas import tpu_sc as plsc

_N = 10000
_E = 320000
_D = 128
_G = 64
_C = 10

_NC = 2
_NS = 16
_NTILES = _NC * _NS
_L = _D // _NTILES

_CH = 128
_SB = 40
_ESB = _SB * _CH
_NSB = 64
_EPAD = _NSB * _ESB
_GRP = _CH // 16

_NACC = 10112
_HR = -(-(_N * _L) // _D)
_AR = (_NACC * _L) // _D


def _sc_agg_body(hs_hbm, src_hbm, dst_hbm, zeros_hbm, out_hbm,
                 h_ts, acc_ts, sbufs, dbufs, sems):
    c = lax.axis_index("c")
    s = lax.axis_index("s")
    t = c * _NS + s

    pltpu.sync_copy(hs_hbm.at[t], h_ts)
    pltpu.sync_copy(zeros_hbm, acc_ts)
    pltpu.sync_copy(src_hbm.at[0], sbufs[0])
    pltpu.sync_copy(dst_hbm.at[0], dbufs[0])

    def pf_start(sb, b):
        pltpu.async_copy(src_hbm.at[sb], sbufs[b], sems[2 * b])
        pltpu.async_copy(dst_hbm.at[sb], dbufs[b], sems[2 * b + 1])

    def pf_wait(sb, b):
        pltpu.make_async_copy(src_hbm.at[sb], sbufs[b], sems[2 * b]).wait()
        pltpu.make_async_copy(dst_hbm.at[sb], dbufs[b], sems[2 * b + 1]).wait()

    def compute(sidx, didx):
        @plsc.parallel_loop(0, _SB * _GRP, unroll=8)
        def _group(gi):
            off = gi * 16
            sv = sidx[pl.ds(off, 16)]
            dv = didx[pl.ds(off, 16)]
            sa = sv << 2
            da = dv << 2
            for l in range(_L):
                v = plsc.load_gather(h_ts, [sa + l])
                plsc.addupdate_scatter(acc_ts, [da + l], v)

    def pair(sb2, carry):
        sb0 = 2 * sb2
        pf_start(sb0 + 1, 1)
        compute(sbufs[0], dbufs[0])
        pf_wait(sb0 + 1, 1)

        @pl.when(sb2 < _NSB // 2 - 1)
        def _():
            pf_start(sb0 + 2, 0)

        compute(sbufs[1], dbufs[1])

        @pl.when(sb2 < _NSB // 2 - 1)
        def _():
            pf_wait(sb0 + 2, 0)

        return carry

    lax.fori_loop(0, _NSB // 2, pair, 0)

    pltpu.sync_copy(acc_ts, out_hbm.at[t])


@functools.partial(jax.jit, static_argnames=())
def _sc_agg(hs, src_pad, dst_pad, zeros_tile):
    mesh = plsc.VectorSubcoreMesh(core_axis_name="c", subcore_axis_name="s")
    return pl.kernel(
        _sc_agg_body,
        out_type=jax.ShapeDtypeStruct((_NTILES, _AR * _D), jnp.float32),
        mesh=mesh,
        compiler_params=pltpu.CompilerParams(needs_layout_passes=False),
        scratch_types=[
            pltpu.VMEM((_HR * _D,), jnp.float32),
            pltpu.VMEM((_AR * _D,), jnp.float32),
            [pltpu.VMEM((_ESB,), jnp.int32) for _ in range(2)],
            [pltpu.VMEM((_ESB,), jnp.int32) for _ in range(2)],
            [pltpu.SemaphoreType.DMA for _ in range(4)],
        ],
    )(hs, src_pad, dst_pad, zeros_tile)


def _pack_h(h):
    hp = h.reshape(_N, _NTILES, _L).transpose(1, 0, 2).reshape(_NTILES, _N * _L)
    return jnp.pad(hp, ((0, 0), (0, _HR * _D - _N * _L)))


def _unpack_agg(out):
    a = out[:, :_N * _L].reshape(_NTILES, _N, _L)
    return a.transpose(1, 0, 2).reshape(_N, _D)


_BN = 1000
_NBLK = _N // _BN


def _mlp_body(x_ref, agg_ref, w1_ref, b1_ref, g_ref, bt_ref, w2_ref, b2_ref,
              batch_ref, h_ref, pool_ref):
    i = pl.program_id(0)
    z = x_ref[...] + agg_ref[...]
    a = jnp.dot(z, w1_ref[...], preferred_element_type=jnp.float32) + b1_ref[...]
    a = a * g_ref[...] + bt_ref[...]
    a = jnp.maximum(a, 0.0)
    h = jnp.dot(a, w2_ref[...], preferred_element_type=jnp.float32) + b2_ref[...]
    h = jnp.maximum(h, 0.0)
    h_ref[...] = h
    gids = batch_ref[...]
    cols = lax.broadcasted_iota(jnp.int32, (_BN, _G), 1)
    onehot = jnp.where(gids == cols, 1.0, 0.0)

    @pl.when(i == 0)
    def _():
        pool_ref[...] = jnp.zeros_like(pool_ref)

    pool_ref[...] += jnp.dot(onehot.T, h, preferred_element_type=jnp.float32)


def _mlp_pool(x, agg, w1, b1, g, bt, w2, b2, batch2d, interpret=False):
    return pl.pallas_call(
        _mlp_body,
        grid=(_NBLK,),
        in_specs=[
            pl.BlockSpec((_BN, _D), lambda i: (i, 0)),
            pl.BlockSpec((_BN, _D), lambda i: (i, 0)),
            pl.BlockSpec((_D, _D), lambda i: (0, 0)),
            pl.BlockSpec((1, _D), lambda i: (0, 0)),
            pl.BlockSpec((1, _D), lambda i: (0, 0)),
            pl.BlockSpec((1, _D), lambda i: (0, 0)),
            pl.BlockSpec((_D, _D), lambda i: (0, 0)),
            pl.BlockSpec((1, _D), lambda i: (0, 0)),
            pl.BlockSpec((_BN, 1), lambda i: (i, 0)),
        ],
        out_specs=[
            pl.BlockSpec((_BN, _D), lambda i: (i, 0)),
            pl.BlockSpec((_G, _D), lambda i: (0, 0)),
        ],
        out_shape=[
            jax.ShapeDtypeStruct((_N, _D), jnp.float32),
            jax.ShapeDtypeStruct((_G, _D), jnp.float32),
        ],
        interpret=interpret,
    )(x, agg, w1, b1, g, bt, w2, b2, batch2d)


def _head_body(p1_ref, p2_ref, p3_ref, w1_ref, b1_ref, w2_ref, b2_ref, o_ref):
    h = jnp.concatenate((p1_ref[...], p2_ref[...], p3_ref[...]), axis=1)
    h = jnp.dot(h, w1_ref[...], preferred_element_type=jnp.float32) + b1_ref[...]
    h = jnp.maximum(h, 0.0)
    logits = jnp.dot(h, w2_ref[...], preferred_element_type=jnp.float32) + b2_ref[...]
    valid = lax.broadcasted_iota(jnp.int32, (_G, _D), 1) < _C
    neg = jnp.float32(-1e30)
    mx = jnp.max(jnp.where(valid, logits, neg), axis=1, keepdims=True)
    ex = jnp.where(valid, jnp.exp(logits - mx), 0.0)
    lse = jnp.log(jnp.sum(ex, axis=1, keepdims=True))
    o_ref[...] = logits - mx - lse


def _head(p1, p2, p3, w1, b1, w2pad, b2pad, interpret=False):
    return pl.pallas_call(
        _head_body,
        out_shape=jax.ShapeDtypeStruct((_G, _D), jnp.float32),
        interpret=interpret,
    )(p1, p2, p3, w1, b1, w2pad, b2pad)


def kernel(x, edge_index, batch, params):
    src = edge_index[0].astype(jnp.int32)
    dst = edge_index[1].astype(jnp.int32)
    npad = _EPAD - _E
    src_pad = jnp.concatenate(
        [src, jnp.zeros((npad,), jnp.int32)]).reshape(_NSB, _ESB)
    dst_pad = jnp.concatenate(
        [dst, _N + (jnp.arange(npad, dtype=jnp.int32) % (_NACC - _N))]
    ).reshape(_NSB, _ESB)
    zeros_tile = jnp.zeros((_AR * _D,), jnp.float32)
    batch2d = batch.astype(jnp.int32).reshape(_N, 1)

    bn_scale = 1.0 / jnp.sqrt(jnp.float32(1.0 + 1e-5))

    def layer(h, l):
        agg = _unpack_agg(_sc_agg(_pack_h(h), src_pad, dst_pad, zeros_tile))
        return _mlp_pool(
            h, agg,
            params[f'c{l}_W1'], params[f'c{l}_b1'].reshape(1, _D),
            (params[f'c{l}_gamma'] * bn_scale).reshape(1, _D),
            params[f'c{l}_beta'].reshape(1, _D),
            params[f'c{l}_W2'], params[f'c{l}_b2'].reshape(1, _D),
            batch2d)

    h1, p1 = layer(x, 1)
    h2, p2 = layer(h1, 2)
    _, p3 = layer(h2, 3)

    w2pad = jnp.zeros((3 * _D, _D), jnp.float32).at[:, :_C].set(params['lin2_W'])
    b2pad = jnp.zeros((1, _D), jnp.float32).at[0, :_C].set(params['lin2_b'])
    out = _head(p1, p2, p3,
                params['lin1_W'], params['lin1_b'].reshape(1, 3 * _D),
                w2pad, b2pad)
    return out[:, :_C]


# speedup vs baseline: 1.0683x; 1.0683x over previous
import functools

import jax
import jax.numpy as jnp
from jax import lax
from jax.experimental import pallas as pl
from jax.experimental.pallas import tpu as pltpu
from jax.experimental.pallas import tpu_sc as plsc

_N = 10000
_E = 320000
_D = 128
_G = 64
_C = 10

_NC = 2
_NS = 16
_NTILES = _NC * _NS
_L = _D // _NTILES

_CH = 128
_SB = 40
_ESB = _SB * _CH
_NSB = 64
_EPAD = _NSB * _ESB
_GRP = _CH // 16

_NACC = 10112
_HR = -(-(_N * _L) // _D)
_AR = (_NACC * _L) // _D


def _sc_agg_body(hs_hbm, src_hbm, dst_hbm, zeros_hbm, out_hbm,
                 h_ts, acc_ts, sbufs, dbufs, sems):
    c = lax.axis_index("c")
    s = lax.axis_index("s")
    t = c * _NS + s

    pltpu.sync_copy(hs_hbm.at[t], h_ts)
    pltpu.sync_copy(zeros_hbm, acc_ts)
    pltpu.sync_copy(src_hbm.at[0], sbufs[0])
    pltpu.sync_copy(dst_hbm.at[0], dbufs[0])

    def pf_start(sb, b):
        pltpu.async_copy(src_hbm.at[sb], sbufs[b], sems[2 * b])
        pltpu.async_copy(dst_hbm.at[sb], dbufs[b], sems[2 * b + 1])

    def pf_wait(sb, b):
        pltpu.make_async_copy(src_hbm.at[sb], sbufs[b], sems[2 * b]).wait()
        pltpu.make_async_copy(dst_hbm.at[sb], dbufs[b], sems[2 * b + 1]).wait()

    def compute(sidx, didx):
        @plsc.parallel_loop(0, _SB * _GRP, unroll=4)
        def _group(gi):
            off = gi * 16
            sv = sidx[pl.ds(off, 16)]
            dv = didx[pl.ds(off, 16)]
            sa = sv << 2
            da = dv << 2
            for l in range(_L):
                v = plsc.load_gather(h_ts, [sa + l])
                plsc.addupdate_scatter(acc_ts, [da + l], v)

    def pair(sb2, carry):
        sb0 = 2 * sb2
        pf_start(sb0 + 1, 1)
        compute(sbufs[0], dbufs[0])
        pf_wait(sb0 + 1, 1)

        @pl.when(sb2 < _NSB // 2 - 1)
        def _():
            pf_start(sb0 + 2, 0)

        compute(sbufs[1], dbufs[1])

        @pl.when(sb2 < _NSB // 2 - 1)
        def _():
            pf_wait(sb0 + 2, 0)

        return carry

    lax.fori_loop(0, _NSB // 2, pair, 0)

    pltpu.sync_copy(acc_ts, out_hbm.at[t])


@functools.partial(jax.jit, static_argnames=())
def _sc_agg(hs, src_pad, dst_pad, zeros_tile):
    mesh = plsc.VectorSubcoreMesh(core_axis_name="c", subcore_axis_name="s")
    return pl.kernel(
        _sc_agg_body,
        out_type=jax.ShapeDtypeStruct((_NTILES, _AR * _D), jnp.float32),
        mesh=mesh,
        compiler_params=pltpu.CompilerParams(needs_layout_passes=False),
        scratch_types=[
            pltpu.VMEM((_HR * _D,), jnp.float32),
            pltpu.VMEM((_AR * _D,), jnp.float32),
            [pltpu.VMEM((_ESB,), jnp.int32) for _ in range(2)],
            [pltpu.VMEM((_ESB,), jnp.int32) for _ in range(2)],
            [pltpu.SemaphoreType.DMA for _ in range(4)],
        ],
    )(hs, src_pad, dst_pad, zeros_tile)


def _pack_h(h):
    hp = h.reshape(_N, _NTILES, _L).transpose(1, 0, 2).reshape(_NTILES, _N * _L)
    return jnp.pad(hp, ((0, 0), (0, _HR * _D - _N * _L)))


def _unpack_agg(out):
    a = out[:, :_N * _L].reshape(_NTILES, _N, _L)
    return a.transpose(1, 0, 2).reshape(_N, _D)


_BN = 1000
_NBLK = _N // _BN


def _mlp_body(x_ref, agg_ref, w1_ref, b1_ref, g_ref, bt_ref, w2_ref, b2_ref,
              batch_ref, h_ref, pool_ref):
    i = pl.program_id(0)
    z = x_ref[...] + agg_ref[...]
    a = jnp.dot(z, w1_ref[...], preferred_element_type=jnp.float32) + b1_ref[...]
    a = a * g_ref[...] + bt_ref[...]
    a = jnp.maximum(a, 0.0)
    h = jnp.dot(a, w2_ref[...], preferred_element_type=jnp.float32) + b2_ref[...]
    h = jnp.maximum(h, 0.0)
    h_ref[...] = h
    gids = batch_ref[...]
    cols = lax.broadcasted_iota(jnp.int32, (_BN, _G), 1)
    onehot = jnp.where(gids == cols, 1.0, 0.0)

    @pl.when(i == 0)
    def _():
        pool_ref[...] = jnp.zeros_like(pool_ref)

    pool_ref[...] += jnp.dot(onehot.T, h, preferred_element_type=jnp.float32)


def _mlp_pool(x, agg, w1, b1, g, bt, w2, b2, batch2d, interpret=False):
    return pl.pallas_call(
        _mlp_body,
        grid=(_NBLK,),
        in_specs=[
            pl.BlockSpec((_BN, _D), lambda i: (i, 0)),
            pl.BlockSpec((_BN, _D), lambda i: (i, 0)),
            pl.BlockSpec((_D, _D), lambda i: (0, 0)),
            pl.BlockSpec((1, _D), lambda i: (0, 0)),
            pl.BlockSpec((1, _D), lambda i: (0, 0)),
            pl.BlockSpec((1, _D), lambda i: (0, 0)),
            pl.BlockSpec((_D, _D), lambda i: (0, 0)),
            pl.BlockSpec((1, _D), lambda i: (0, 0)),
            pl.BlockSpec((_BN, 1), lambda i: (i, 0)),
        ],
        out_specs=[
            pl.BlockSpec((_BN, _D), lambda i: (i, 0)),
            pl.BlockSpec((_G, _D), lambda i: (0, 0)),
        ],
        out_shape=[
            jax.ShapeDtypeStruct((_N, _D), jnp.float32),
            jax.ShapeDtypeStruct((_G, _D), jnp.float32),
        ],
        interpret=interpret,
    )(x, agg, w1, b1, g, bt, w2, b2, batch2d)


def _head_body(p1_ref, p2_ref, p3_ref, w1_ref, b1_ref, w2_ref, b2_ref, o_ref):
    h = jnp.concatenate((p1_ref[...], p2_ref[...], p3_ref[...]), axis=1)
    h = jnp.dot(h, w1_ref[...], preferred_element_type=jnp.float32) + b1_ref[...]
    h = jnp.maximum(h, 0.0)
    logits = jnp.dot(h, w2_ref[...], preferred_element_type=jnp.float32) + b2_ref[...]
    valid = lax.broadcasted_iota(jnp.int32, (_G, _D), 1) < _C
    neg = jnp.float32(-1e30)
    mx = jnp.max(jnp.where(valid, logits, neg), axis=1, keepdims=True)
    ex = jnp.where(valid, jnp.exp(logits - mx), 0.0)
    lse = jnp.log(jnp.sum(ex, axis=1, keepdims=True))
    o_ref[...] = logits - mx - lse


def _head(p1, p2, p3, w1, b1, w2pad, b2pad, interpret=False):
    return pl.pallas_call(
        _head_body,
        out_shape=jax.ShapeDtypeStruct((_G, _D), jnp.float32),
        interpret=interpret,
    )(p1, p2, p3, w1, b1, w2pad, b2pad)


def kernel(x, edge_index, batch, params):
    src = edge_index[0].astype(jnp.int32)
    dst = edge_index[1].astype(jnp.int32)
    npad = _EPAD - _E
    src_pad = jnp.concatenate(
        [src, jnp.zeros((npad,), jnp.int32)]).reshape(_NSB, _ESB)
    dst_pad = jnp.concatenate(
        [dst, _N + (jnp.arange(npad, dtype=jnp.int32) % (_NACC - _N))]
    ).reshape(_NSB, _ESB)
    zeros_tile = jnp.zeros((_AR * _D,), jnp.float32)
    batch2d = batch.astype(jnp.int32).reshape(_N, 1)

    bn_scale = 1.0 / jnp.sqrt(jnp.float32(1.0 + 1e-5))

    def layer(h, l):
        agg = _unpack_agg(_sc_agg(_pack_h(h), src_pad, dst_pad, zeros_tile))
        return _mlp_pool(
            h, agg,
            params[f'c{l}_W1'], params[f'c{l}_b1'].reshape(1, _D),
            (params[f'c{l}_gamma'] * bn_scale).reshape(1, _D),
            params[f'c{l}_beta'].reshape(1, _D),
            params[f'c{l}_W2'], params[f'c{l}_b2'].reshape(1, _D),
            batch2d)

    h1, p1 = layer(x, 1)
    h2, p2 = layer(h1, 2)
    _, p3 = layer(h2, 3)

    w2pad = jnp.zeros((3 * _D, _D), jnp.float32).at[:, :_C].set(params['lin2_W'])
    b2pad = jnp.zeros((1, _D), jnp.float32).at[0, :_C].set(params['lin2_b'])
    out = _head(p1, p2, p3,
                params['lin1_W'], params['lin1_b'].reshape(1, 3 * _D),
                w2pad, b2pad)
    return out[:, :_C]


# speedup vs baseline: 1.0981x; 1.0279x over previous
import functools

import jax
import jax.numpy as jnp
from jax import lax
from jax.experimental import pallas as pl
from jax.experimental.pallas import tpu as pltpu
from jax.experimental.pallas import tpu_sc as plsc

_N = 10000
_E = 320000
_D = 128
_G = 64
_C = 10

_NC = 2
_NS = 16
_NTILES = _NC * _NS
_L = _D // _NTILES

_CH = 128
_SB = 40
_ESB = _SB * _CH
_NSB = 64
_EPAD = _NSB * _ESB
_GRP = _CH // 16

_NACC = 10112
_HR = -(-(_N * _L) // _D)
_AR = (_NACC * _L) // _D


def _sc_agg_body(hs_hbm, src_hbm, dst_hbm, zeros_hbm, out_hbm,
                 h_ts, acc_ts, sbufs, dbufs, sems):
    c = lax.axis_index("c")
    s = lax.axis_index("s")
    t = c * _NS + s

    pltpu.sync_copy(hs_hbm.at[t], h_ts)
    pltpu.sync_copy(zeros_hbm, acc_ts)
    pltpu.sync_copy(src_hbm.at[0], sbufs[0])
    pltpu.sync_copy(dst_hbm.at[0], dbufs[0])

    def pf_start(sb, b):
        pltpu.async_copy(src_hbm.at[sb], sbufs[b], sems[2 * b])
        pltpu.async_copy(dst_hbm.at[sb], dbufs[b], sems[2 * b + 1])

    def pf_wait(sb, b):
        pltpu.make_async_copy(src_hbm.at[sb], sbufs[b], sems[2 * b]).wait()
        pltpu.make_async_copy(dst_hbm.at[sb], dbufs[b], sems[2 * b + 1]).wait()

    def compute(sidx, didx):
        @plsc.parallel_loop(0, _SB * _GRP, unroll=2)
        def _group(gi):
            off = gi * 16
            sv = sidx[pl.ds(off, 16)]
            dv = didx[pl.ds(off, 16)]
            sa = sv << 2
            da = dv << 2
            for l in range(_L):
                v = plsc.load_gather(h_ts, [sa + l])
                plsc.addupdate_scatter(acc_ts, [da + l], v)

    def pair(sb2, carry):
        sb0 = 2 * sb2
        pf_start(sb0 + 1, 1)
        compute(sbufs[0], dbufs[0])
        pf_wait(sb0 + 1, 1)

        @pl.when(sb2 < _NSB // 2 - 1)
        def _():
            pf_start(sb0 + 2, 0)

        compute(sbufs[1], dbufs[1])

        @pl.when(sb2 < _NSB // 2 - 1)
        def _():
            pf_wait(sb0 + 2, 0)

        return carry

    lax.fori_loop(0, _NSB // 2, pair, 0)

    pltpu.sync_copy(acc_ts, out_hbm.at[t])


@functools.partial(jax.jit, static_argnames=())
def _sc_agg(hs, src_pad, dst_pad, zeros_tile):
    mesh = plsc.VectorSubcoreMesh(core_axis_name="c", subcore_axis_name="s")
    return pl.kernel(
        _sc_agg_body,
        out_type=jax.ShapeDtypeStruct((_NTILES, _AR * _D), jnp.float32),
        mesh=mesh,
        compiler_params=pltpu.CompilerParams(needs_layout_passes=False),
        scratch_types=[
            pltpu.VMEM((_HR * _D,), jnp.float32),
            pltpu.VMEM((_AR * _D,), jnp.float32),
            [pltpu.VMEM((_ESB,), jnp.int32) for _ in range(2)],
            [pltpu.VMEM((_ESB,), jnp.int32) for _ in range(2)],
            [pltpu.SemaphoreType.DMA for _ in range(4)],
        ],
    )(hs, src_pad, dst_pad, zeros_tile)


def _pack_h(h):
    hp = h.reshape(_N, _NTILES, _L).transpose(1, 0, 2).reshape(_NTILES, _N * _L)
    return jnp.pad(hp, ((0, 0), (0, _HR * _D - _N * _L)))


def _unpack_agg(out):
    a = out[:, :_N * _L].reshape(_NTILES, _N, _L)
    return a.transpose(1, 0, 2).reshape(_N, _D)


_BN = 1000
_NBLK = _N // _BN


def _mlp_body(x_ref, agg_ref, w1_ref, b1_ref, g_ref, bt_ref, w2_ref, b2_ref,
              batch_ref, h_ref, pool_ref):
    i = pl.program_id(0)
    z = x_ref[...] + agg_ref[...]
    a = jnp.dot(z, w1_ref[...], preferred_element_type=jnp.float32) + b1_ref[...]
    a = a * g_ref[...] + bt_ref[...]
    a = jnp.maximum(a, 0.0)
    h = jnp.dot(a, w2_ref[...], preferred_element_type=jnp.float32) + b2_ref[...]
    h = jnp.maximum(h, 0.0)
    h_ref[...] = h
    gids = batch_ref[...]
    cols = lax.broadcasted_iota(jnp.int32, (_BN, _G), 1)
    onehot = jnp.where(gids == cols, 1.0, 0.0)

    @pl.when(i == 0)
    def _():
        pool_ref[...] = jnp.zeros_like(pool_ref)

    pool_ref[...] += jnp.dot(onehot.T, h, preferred_element_type=jnp.float32)


def _mlp_pool(x, agg, w1, b1, g, bt, w2, b2, batch2d, interpret=False):
    return pl.pallas_call(
        _mlp_body,
        grid=(_NBLK,),
        in_specs=[
            pl.BlockSpec((_BN, _D), lambda i: (i, 0)),
            pl.BlockSpec((_BN, _D), lambda i: (i, 0)),
            pl.BlockSpec((_D, _D), lambda i: (0, 0)),
            pl.BlockSpec((1, _D), lambda i: (0, 0)),
            pl.BlockSpec((1, _D), lambda i: (0, 0)),
            pl.BlockSpec((1, _D), lambda i: (0, 0)),
            pl.BlockSpec((_D, _D), lambda i: (0, 0)),
            pl.BlockSpec((1, _D), lambda i: (0, 0)),
            pl.BlockSpec((_BN, 1), lambda i: (i, 0)),
        ],
        out_specs=[
            pl.BlockSpec((_BN, _D), lambda i: (i, 0)),
            pl.BlockSpec((_G, _D), lambda i: (0, 0)),
        ],
        out_shape=[
            jax.ShapeDtypeStruct((_N, _D), jnp.float32),
            jax.ShapeDtypeStruct((_G, _D), jnp.float32),
        ],
        interpret=interpret,
    )(x, agg, w1, b1, g, bt, w2, b2, batch2d)


def _head_body(p1_ref, p2_ref, p3_ref, w1_ref, b1_ref, w2_ref, b2_ref, o_ref):
    h = jnp.concatenate((p1_ref[...], p2_ref[...], p3_ref[...]), axis=1)
    h = jnp.dot(h, w1_ref[...], preferred_element_type=jnp.float32) + b1_ref[...]
    h = jnp.maximum(h, 0.0)
    logits = jnp.dot(h, w2_ref[...], preferred_element_type=jnp.float32) + b2_ref[...]
    valid = lax.broadcasted_iota(jnp.int32, (_G, _D), 1) < _C
    neg = jnp.float32(-1e30)
    mx = jnp.max(jnp.where(valid, logits, neg), axis=1, keepdims=True)
    ex = jnp.where(valid, jnp.exp(logits - mx), 0.0)
    lse = jnp.log(jnp.sum(ex, axis=1, keepdims=True))
    o_ref[...] = logits - mx - lse


def _head(p1, p2, p3, w1, b1, w2pad, b2pad, interpret=False):
    return pl.pallas_call(
        _head_body,
        out_shape=jax.ShapeDtypeStruct((_G, _D), jnp.float32),
        interpret=interpret,
    )(p1, p2, p3, w1, b1, w2pad, b2pad)


def kernel(x, edge_index, batch, params):
    src = edge_index[0].astype(jnp.int32)
    dst = edge_index[1].astype(jnp.int32)
    npad = _EPAD - _E
    src_pad = jnp.concatenate(
        [src, jnp.zeros((npad,), jnp.int32)]).reshape(_NSB, _ESB)
    dst_pad = jnp.concatenate(
        [dst, _N + (jnp.arange(npad, dtype=jnp.int32) % (_NACC - _N))]
    ).reshape(_NSB, _ESB)
    zeros_tile = jnp.zeros((_AR * _D,), jnp.float32)
    batch2d = batch.astype(jnp.int32).reshape(_N, 1)

    bn_scale = 1.0 / jnp.sqrt(jnp.float32(1.0 + 1e-5))

    def layer(h, l):
        agg = _unpack_agg(_sc_agg(_pack_h(h), src_pad, dst_pad, zeros_tile))
        return _mlp_pool(
            h, agg,
            params[f'c{l}_W1'], params[f'c{l}_b1'].reshape(1, _D),
            (params[f'c{l}_gamma'] * bn_scale).reshape(1, _D),
            params[f'c{l}_beta'].reshape(1, _D),
            params[f'c{l}_W2'], params[f'c{l}_b2'].reshape(1, _D),
            batch2d)

    h1, p1 = layer(x, 1)
    h2, p2 = layer(h1, 2)
    _, p3 = layer(h2, 3)

    w2pad = jnp.zeros((3 * _D, _D), jnp.float32).at[:, :_C].set(params['lin2_W'])
    b2pad = jnp.zeros((1, _D), jnp.float32).at[0, :_C].set(params['lin2_b'])
    out = _head(p1, p2, p3,
                params['lin1_W'], params['lin1_b'].reshape(1, 3 * _D),
                w2pad, b2pad)
    return out[:, :_C]


# speedup vs baseline: 1.1332x; 1.0319x over previous
import functools

import jax
import jax.numpy as jnp
from jax import lax
from jax.experimental import pallas as pl
from jax.experimental.pallas import tpu as pltpu
from jax.experimental.pallas import tpu_sc as plsc

_N = 10000
_E = 320000
_D = 128
_G = 64
_C = 10

_NC = 2
_NS = 16
_NTILES = _NC * _NS
_L = _D // _NTILES

_CH = 128
_SB = 40
_ESB = _SB * _CH
_NSB = 64
_EPAD = _NSB * _ESB
_GRP = _CH // 16

_NACC = 10112
_HR = -(-(_N * _L) // _D)
_AR = (_NACC * _L) // _D


def _sc_agg_body(hs_hbm, src_hbm, dst_hbm, zeros_hbm, out_hbm,
                 h_ts, acc_ts, sbufs, dbufs, sems):
    c = lax.axis_index("c")
    s = lax.axis_index("s")
    t = c * _NS + s

    pltpu.sync_copy(hs_hbm.at[t], h_ts)
    pltpu.sync_copy(zeros_hbm, acc_ts)
    pltpu.sync_copy(src_hbm.at[0], sbufs[0])
    pltpu.sync_copy(dst_hbm.at[0], dbufs[0])

    def pf_start(sb, b):
        pltpu.async_copy(src_hbm.at[sb], sbufs[b], sems[2 * b])
        pltpu.async_copy(dst_hbm.at[sb], dbufs[b], sems[2 * b + 1])

    def pf_wait(sb, b):
        pltpu.make_async_copy(src_hbm.at[sb], sbufs[b], sems[2 * b]).wait()
        pltpu.make_async_copy(dst_hbm.at[sb], dbufs[b], sems[2 * b + 1]).wait()

    def compute(sidx, didx):
        @plsc.parallel_loop(0, _SB * _GRP, unroll=1)
        def _group(gi):
            off = gi * 16
            sv = sidx[pl.ds(off, 16)]
            dv = didx[pl.ds(off, 16)]
            sa = sv << 2
            da = dv << 2
            for l in range(_L):
                v = plsc.load_gather(h_ts, [sa + l])
                plsc.addupdate_scatter(acc_ts, [da + l], v)

    def pair(sb2, carry):
        sb0 = 2 * sb2
        pf_start(sb0 + 1, 1)
        compute(sbufs[0], dbufs[0])
        pf_wait(sb0 + 1, 1)

        @pl.when(sb2 < _NSB // 2 - 1)
        def _():
            pf_start(sb0 + 2, 0)

        compute(sbufs[1], dbufs[1])

        @pl.when(sb2 < _NSB // 2 - 1)
        def _():
            pf_wait(sb0 + 2, 0)

        return carry

    lax.fori_loop(0, _NSB // 2, pair, 0)

    pltpu.sync_copy(acc_ts, out_hbm.at[t])


@functools.partial(jax.jit, static_argnames=())
def _sc_agg(hs, src_pad, dst_pad, zeros_tile):
    mesh = plsc.VectorSubcoreMesh(core_axis_name="c", subcore_axis_name="s")
    return pl.kernel(
        _sc_agg_body,
        out_type=jax.ShapeDtypeStruct((_NTILES, _AR * _D), jnp.float32),
        mesh=mesh,
        compiler_params=pltpu.CompilerParams(needs_layout_passes=False),
        scratch_types=[
            pltpu.VMEM((_HR * _D,), jnp.float32),
            pltpu.VMEM((_AR * _D,), jnp.float32),
            [pltpu.VMEM((_ESB,), jnp.int32) for _ in range(2)],
            [pltpu.VMEM((_ESB,), jnp.int32) for _ in range(2)],
            [pltpu.SemaphoreType.DMA for _ in range(4)],
        ],
    )(hs, src_pad, dst_pad, zeros_tile)


def _pack_h(h):
    hp = h.reshape(_N, _NTILES, _L).transpose(1, 0, 2).reshape(_NTILES, _N * _L)
    return jnp.pad(hp, ((0, 0), (0, _HR * _D - _N * _L)))


def _unpack_agg(out):
    a = out[:, :_N * _L].reshape(_NTILES, _N, _L)
    return a.transpose(1, 0, 2).reshape(_N, _D)


_BN = 1000
_NBLK = _N // _BN


def _mlp_body(x_ref, agg_ref, w1_ref, b1_ref, g_ref, bt_ref, w2_ref, b2_ref,
              batch_ref, h_ref, pool_ref):
    i = pl.program_id(0)
    z = x_ref[...] + agg_ref[...]
    a = jnp.dot(z, w1_ref[...], preferred_element_type=jnp.float32) + b1_ref[...]
    a = a * g_ref[...] + bt_ref[...]
    a = jnp.maximum(a, 0.0)
    h = jnp.dot(a, w2_ref[...], preferred_element_type=jnp.float32) + b2_ref[...]
    h = jnp.maximum(h, 0.0)
    h_ref[...] = h
    gids = batch_ref[...]
    cols = lax.broadcasted_iota(jnp.int32, (_BN, _G), 1)
    onehot = jnp.where(gids == cols, 1.0, 0.0)

    @pl.when(i == 0)
    def _():
        pool_ref[...] = jnp.zeros_like(pool_ref)

    pool_ref[...] += jnp.dot(onehot.T, h, preferred_element_type=jnp.float32)


def _mlp_pool(x, agg, w1, b1, g, bt, w2, b2, batch2d, interpret=False):
    return pl.pallas_call(
        _mlp_body,
        grid=(_NBLK,),
        in_specs=[
            pl.BlockSpec((_BN, _D), lambda i: (i, 0)),
            pl.BlockSpec((_BN, _D), lambda i: (i, 0)),
            pl.BlockSpec((_D, _D), lambda i: (0, 0)),
            pl.BlockSpec((1, _D), lambda i: (0, 0)),
            pl.BlockSpec((1, _D), lambda i: (0, 0)),
            pl.BlockSpec((1, _D), lambda i: (0, 0)),
            pl.BlockSpec((_D, _D), lambda i: (0, 0)),
            pl.BlockSpec((1, _D), lambda i: (0, 0)),
            pl.BlockSpec((_BN, 1), lambda i: (i, 0)),
        ],
        out_specs=[
            pl.BlockSpec((_BN, _D), lambda i: (i, 0)),
            pl.BlockSpec((_G, _D), lambda i: (0, 0)),
        ],
        out_shape=[
            jax.ShapeDtypeStruct((_N, _D), jnp.float32),
            jax.ShapeDtypeStruct((_G, _D), jnp.float32),
        ],
        interpret=interpret,
    )(x, agg, w1, b1, g, bt, w2, b2, batch2d)


def _head_body(p1_ref, p2_ref, p3_ref, w1_ref, b1_ref, w2_ref, b2_ref, o_ref):
    h = jnp.concatenate((p1_ref[...], p2_ref[...], p3_ref[...]), axis=1)
    h = jnp.dot(h, w1_ref[...], preferred_element_type=jnp.float32) + b1_ref[...]
    h = jnp.maximum(h, 0.0)
    logits = jnp.dot(h, w2_ref[...], preferred_element_type=jnp.float32) + b2_ref[...]
    valid = lax.broadcasted_iota(jnp.int32, (_G, _D), 1) < _C
    neg = jnp.float32(-1e30)
    mx = jnp.max(jnp.where(valid, logits, neg), axis=1, keepdims=True)
    ex = jnp.where(valid, jnp.exp(logits - mx), 0.0)
    lse = jnp.log(jnp.sum(ex, axis=1, keepdims=True))
    o_ref[...] = logits - mx - lse


def _head(p1, p2, p3, w1, b1, w2pad, b2pad, interpret=False):
    return pl.pallas_call(
        _head_body,
        out_shape=jax.ShapeDtypeStruct((_G, _D), jnp.float32),
        interpret=interpret,
    )(p1, p2, p3, w1, b1, w2pad, b2pad)


def kernel(x, edge_index, batch, params):
    src = edge_index[0].astype(jnp.int32)
    dst = edge_index[1].astype(jnp.int32)
    npad = _EPAD - _E
    src_pad = jnp.concatenate(
        [src, jnp.zeros((npad,), jnp.int32)]).reshape(_NSB, _ESB)
    dst_pad = jnp.concatenate(
        [dst, _N + (jnp.arange(npad, dtype=jnp.int32) % (_NACC - _N))]
    ).reshape(_NSB, _ESB)
    zeros_tile = jnp.zeros((_AR * _D,), jnp.float32)
    batch2d = batch.astype(jnp.int32).reshape(_N, 1)

    bn_scale = 1.0 / jnp.sqrt(jnp.float32(1.0 + 1e-5))

    def layer(h, l):
        agg = _unpack_agg(_sc_agg(_pack_h(h), src_pad, dst_pad, zeros_tile))
        return _mlp_pool(
            h, agg,
            params[f'c{l}_W1'], params[f'c{l}_b1'].reshape(1, _D),
            (params[f'c{l}_gamma'] * bn_scale).reshape(1, _D),
            params[f'c{l}_beta'].reshape(1, _D),
            params[f'c{l}_W2'], params[f'c{l}_b2'].reshape(1, _D),
            batch2d)

    h1, p1 = layer(x, 1)
    h2, p2 = layer(h1, 2)
    _, p3 = layer(h2, 3)

    w2pad = jnp.zeros((3 * _D, _D), jnp.float32).at[:, :_C].set(params['lin2_W'])
    b2pad = jnp.zeros((1, _D), jnp.float32).at[0, :_C].set(params['lin2_b'])
    out = _head(p1, p2, p3,
                params['lin1_W'], params['lin1_b'].reshape(1, 3 * _D),
                w2pad, b2pad)
    return out[:, :_C]
